# Initial kernel scaffold; baseline (speedup 1.0000x reference)
import jax
import jax.numpy as jnp
from jax.experimental import pallas as pl


def kernel(X, scores):
    raise NotImplementedError("write your pallas kernel here")


# speedup vs baseline: 19.1642x; 19.1642x over previous
import jax
import jax.numpy as jnp
from jax.experimental import pallas as pl
from jax.experimental.pallas import tpu as pltpu

_B = 128
_N = 8192
_K = 64
_OUT = 16
_TEMP = 0.5


def _gate_kernel(scores_ref, out_ref, gate_ref):
    @pl.when(pl.program_id(0) == 0)
    def _compute_gate():
        st = scores_ref[...].T
        bits = jax.lax.bitcast_convert_type(st, jnp.int32)

        lo = jnp.zeros((_OUT, 1), jnp.int32)
        hi = jnp.full((_OUT, 1), 0x7F800000, jnp.int32)

        def search_body(_, carry):
            lo, hi = carry
            mid = (lo + hi) >> 1
            cnt = jnp.sum((bits >= mid).astype(jnp.int32), axis=1,
                          keepdims=True)
            ge = cnt >= _K
            return jnp.where(ge, mid, lo), jnp.where(ge, hi, mid)

        lo, hi = jax.lax.fori_loop(0, 31, search_body, (lo, hi))
        tau = lo

        gt = bits > tau
        n_gt = jnp.sum(gt.astype(jnp.int32), axis=1, keepdims=True)
        need = _K - n_gt
        tie = bits == tau
        idx = jax.lax.broadcasted_iota(jnp.int32, (_OUT, _N), 1)

        lo2 = jnp.zeros((_OUT, 1), jnp.int32)
        hi2 = jnp.full((_OUT, 1), _N + 1, jnp.int32)

        def tie_body(_, carry):
            lo, hi = carry
            mid = (lo + hi) >> 1
            cnt = jnp.sum((tie & (idx < mid)).astype(jnp.int32), axis=1,
                          keepdims=True)
            ok = cnt <= need
            return jnp.where(ok, mid, lo), jnp.where(ok, hi, mid)

        lo2, hi2 = jax.lax.fori_loop(0, 14, tie_body, (lo2, hi2))
        keep = gt | (tie & (idx < lo2))

        m = jnp.max(st, axis=1, keepdims=True)
        e = jnp.where(keep, jnp.exp((st - m) / _TEMP), 0.0)
        gate_ref[...] = (e / jnp.sum(e, axis=1, keepdims=True)).T

    out_ref[...] = gate_ref[...][None]


def kernel(X, scores):
    del X
    return pl.pallas_call(
        _gate_kernel,
        grid=(_B,),
        in_specs=[pl.BlockSpec((_N, _OUT), lambda b: (0, 0))],
        out_specs=pl.BlockSpec((1, _N, _OUT), lambda b: (b, 0, 0)),
        out_shape=jax.ShapeDtypeStruct((_B, _N, _OUT), jnp.float32),
        scratch_shapes=[pltpu.VMEM((_N, _OUT), jnp.float32)],
    )(scores)
